# baseline (speedup 1.0000x reference)
import functools

import jax
import jax.numpy as jnp
from jax import lax
from jax.experimental import pallas as pl
from jax.experimental.pallas import tpu as pltpu
from jax.experimental.pallas import tpu_sc as plsc

B, N, K, H = 4, 1024, 36, 128
NB = N * K
BN = 32
EB = BN * K

_INV_K = 1.0 / 36.0
_SQRT_HALF = 0.7071067811865476


def _gelu(x):
    c = 0.7978845608028654
    return 0.5 * x * (1.0 + jnp.tanh(c * (x + 0.044715 * x * x * x)))


def _proj_body(hv_ref, w1a_ref, w1c_ref, e_ref, p_ref, q_ref, idx_ref):
    x = hv_ref[...]
    p_ref[...] = jnp.dot(x, w1a_ref[...], preferred_element_type=jnp.float32)
    q_ref[...] = jnp.dot(x, w1c_ref[...], preferred_element_type=jnp.float32)
    off = pl.program_id(0) * N
    idx_ref[...] = e_ref[...] + off


def _project(hVf, W1a, W1c, E2):
    blk = 1024
    return pl.pallas_call(
        _proj_body,
        grid=(B * N // blk,),
        in_specs=[
            pl.BlockSpec((blk, H), lambda i: (i, 0)),
            pl.BlockSpec((H, H), lambda i: (0, 0)),
            pl.BlockSpec((H, H), lambda i: (0, 0)),
            pl.BlockSpec((blk, K), lambda i: (i, 0)),
        ],
        out_specs=[
            pl.BlockSpec((blk, H), lambda i: (i, 0)),
            pl.BlockSpec((blk, H), lambda i: (i, 0)),
            pl.BlockSpec((blk, K), lambda i: (i, 0)),
        ],
        out_shape=[
            jax.ShapeDtypeStruct((B * N, H), jnp.float32),
            jax.ShapeDtypeStruct((B * N, H), jnp.float32),
            jax.ShapeDtypeStruct((B * N, K), jnp.int32),
        ],
    )(hVf, W1a, W1c, E2)


_NW = 32
_NPW = N // _NW
_NBUF = 4


def _make_gather(b):
    mesh = plsc.VectorSubcoreMesh(core_axis_name="c", subcore_axis_name="s")

    @functools.partial(
        pl.kernel,
        out_type=jax.ShapeDtypeStruct((N, K, H), jnp.float32),
        mesh=mesh,
        scratch_types=[
            pltpu.VMEM((_NPW, K), jnp.int32),
            pltpu.VMEM((_NBUF, K, H), jnp.float32),
            pltpu.SemaphoreType.DMA,
            pltpu.SemaphoreType.DMA,
        ],
    )
    def gather_kernel(q_hbm, idx_hbm, out_hbm, idx_v, rows_v, gsem, wsem):
        wid = lax.axis_index("s") * 2 + lax.axis_index("c")
        n0 = wid * _NPW
        pltpu.sync_copy(idx_hbm.at[pl.ds(b * N + n0, _NPW)], idx_v)

        gh = [None] * _NBUF
        wh = [None] * _NBUF
        la = _NBUF - 1
        for i in range(-la, _NPW):
            j = i + la
            if j < _NPW:
                bj = j % _NBUF
                if wh[bj] is not None:
                    wh[bj].wait()
                gh[bj] = pltpu.async_copy(q_hbm.at[idx_v.at[j]],
                                          rows_v.at[bj], gsem)
            if i >= 0:
                bi = i % _NBUF
                gh[bi].wait()
                wh[bi] = pltpu.async_copy(rows_v.at[bi],
                                          out_hbm.at[n0 + i], wsem)
        for i in range(_NPW - _NBUF, _NPW):
            wh[i % _NBUF].wait()

    return gather_kernel


def _dot3(x, w):
    return lax.dot_general(x, w, (((x.ndim - 1,), (0,)), ((), ())),
                           preferred_element_type=jnp.float32)


def _main_body(prev_ref, hE_ref, g_ref, p_ref, hv_ref, ma_ref,
               w1b_ref, b1_ref, w2_ref, b2_ref, w3_ref, b3_ref,
               g1_ref, be1_ref, out_ref):
    del prev_ref
    x = hE_ref[...]
    e1 = _dot3(x, w1b_ref[...])
    pb = p_ref[...][:, None, :]
    t1 = _gelu(e1 + pb + g_ref[...] + b1_ref[...][None])
    t2 = _gelu(_dot3(t1, w2_ref[...]) + b2_ref[...][None])
    ma = ma_ref[...]
    t2m = t2 * ma[:, :, None]
    s = jnp.sum(t2m, axis=1)
    m = jnp.sum(ma, axis=1, keepdims=True)
    dh = (jnp.dot(s, w3_ref[...], preferred_element_type=jnp.float32)
          + m * b3_ref[...]) * _INV_K
    r = hv_ref[...] + dh
    mu = jnp.mean(r, axis=-1, keepdims=True)
    var = jnp.mean((r - mu) ** 2, axis=-1, keepdims=True)
    out_ref[...] = (r - mu) * lax.rsqrt(var + 1e-5) * g1_ref[...] + be1_ref[...]


def _main(b, prev, hE3, G_b, P, hVf, ma2, W1b, b1r, W2, b2r, W3, b3r,
          g1r, be1r):
    nblk = N // BN
    wspec = pl.BlockSpec((H, H), lambda i: (0, 0))
    bspec = pl.BlockSpec((1, H), lambda i: (0, 0))
    return pl.pallas_call(
        _main_body,
        grid=(nblk,),
        in_specs=[
            pl.BlockSpec(memory_space=pl.ANY),
            pl.BlockSpec((BN, K, H), lambda i: (b * nblk + i, 0, 0)),
            pl.BlockSpec((BN, K, H), lambda i: (i, 0, 0)),
            pl.BlockSpec((BN, H), lambda i: (b * nblk + i, 0)),
            pl.BlockSpec((BN, H), lambda i: (b * nblk + i, 0)),
            pl.BlockSpec((BN, K), lambda i: (b * nblk + i, 0)),
            wspec, bspec, wspec, bspec, wspec, bspec,
            bspec, bspec,
        ],
        out_specs=pl.BlockSpec((BN, H), lambda i: (b * nblk + i, 0)),
        out_shape=jax.ShapeDtypeStruct((B * N, H), jnp.float32),
        input_output_aliases={0: 0},
    )(prev, hE3, G_b, P, hVf, ma2, W1b, b1r, W2, b2r, W3, b3r, g1r, be1r)


def _ffn_body(x_ref, win_ref, bi_ref, wout_ref, bo_ref, g2_ref, be2_ref,
              mv_ref, out_ref):
    x = x_ref[...]
    t = _gelu(jnp.dot(x, win_ref[...], preferred_element_type=jnp.float32)
              + bi_ref[...])
    f = jnp.dot(t, wout_ref[...], preferred_element_type=jnp.float32) + bo_ref[...]
    r = x + f
    mu = jnp.mean(r, axis=-1, keepdims=True)
    var = jnp.mean((r - mu) ** 2, axis=-1, keepdims=True)
    out_ref[...] = ((r - mu) * lax.rsqrt(var + 1e-5) * g2_ref[...]
                    + be2_ref[...]) * mv_ref[...]


def _ffn(hv1, Win, bir, Wout, bor, g2r, be2r, mVf):
    blk = 512
    bspec = pl.BlockSpec((1, 4 * H), lambda i: (0, 0))
    return pl.pallas_call(
        _ffn_body,
        grid=(B * N // blk,),
        in_specs=[
            pl.BlockSpec((blk, H), lambda i: (i, 0)),
            pl.BlockSpec((H, 4 * H), lambda i: (0, 0)),
            bspec,
            pl.BlockSpec((4 * H, H), lambda i: (0, 0)),
            pl.BlockSpec((1, H), lambda i: (0, 0)),
            pl.BlockSpec((1, H), lambda i: (0, 0)),
            pl.BlockSpec((1, H), lambda i: (0, 0)),
            pl.BlockSpec((blk, 1), lambda i: (i, 0)),
        ],
        out_specs=pl.BlockSpec((blk, H), lambda i: (i, 0)),
        out_shape=jax.ShapeDtypeStruct((B * N, H), jnp.float32),
    )(hv1, Win, bir, Wout, bor, g2r, be2r, mVf)


def kernel(h_V, h_E, E_idx, mask_V, mask_attend, W1, b1, W2, b2, W3, b3,
           Win, bi, Wout, bo, g1, be1, g2, be2):
    hVf = h_V.reshape(B * N, H)
    W1a, W1b, W1c = W1[:H], W1[H:2 * H], W1[2 * H:]
    b1r, b2r, b3r = b1.reshape(1, H), b2.reshape(1, H), b3.reshape(1, H)
    g1r, be1r = g1.reshape(1, H), be1.reshape(1, H)
    g2r, be2r = g2.reshape(1, H), be2.reshape(1, H)
    bir, bor = bi.reshape(1, 4 * H), bo.reshape(1, H)

    P, Q, idxG = _project(hVf, W1a, W1c, E_idx.reshape(B * N, K))
    hE3 = h_E.reshape(B * N, K, H)
    ma2 = mask_attend.reshape(B * N, K)

    hv1 = jnp.zeros((B * N, H), jnp.float32)
    for b in range(B):
        G_b = _make_gather(b)(Q, idxG)
        hv1 = _main(b, hv1, hE3, G_b, P, hVf, ma2,
                    W1b, b1r, W2, b2r, W3, b3r, g1r, be1r)

    out = _ffn(hv1, Win, bir, Wout, bor, g2r, be2r, mask_V.reshape(B * N, 1))
    return out.reshape(B, N, H)


# speedup vs baseline: 1.0775x; 1.0775x over previous
import functools

import jax
import jax.numpy as jnp
from jax import lax
from jax.experimental import pallas as pl
from jax.experimental.pallas import tpu as pltpu
from jax.experimental.pallas import tpu_sc as plsc

B, N, K, H = 4, 1024, 36, 128
NB = N * K
BN = 32
EB = BN * K

_INV_K = 1.0 / 36.0
_SQRT_HALF = 0.7071067811865476


def _gelu(x):
    return x * 0.5 * (1.0 + lax.erf(x * _SQRT_HALF))


def _proj_body(hv_ref, w1a_ref, w1c_ref, e_ref, p_ref, q_ref, idx_ref):
    x = hv_ref[...]
    p_ref[...] = jnp.dot(x, w1a_ref[...], preferred_element_type=jnp.float32)
    q_ref[...] = jnp.dot(x, w1c_ref[...], preferred_element_type=jnp.float32)
    off = pl.program_id(0) * N
    ev = e_ref[...] + off
    idx_ref[...] = jnp.concatenate(
        [ev, jnp.zeros((ev.shape[0], 128 - K), jnp.int32)], axis=1)


def _project(hVf, W1a, W1c, E2):
    blk = 1024
    return pl.pallas_call(
        _proj_body,
        grid=(B * N // blk,),
        in_specs=[
            pl.BlockSpec((blk, H), lambda i: (i, 0)),
            pl.BlockSpec((H, H), lambda i: (0, 0)),
            pl.BlockSpec((H, H), lambda i: (0, 0)),
            pl.BlockSpec((blk, K), lambda i: (i, 0)),
        ],
        out_specs=[
            pl.BlockSpec((blk, H), lambda i: (i, 0)),
            pl.BlockSpec((blk, H), lambda i: (i, 0)),
            pl.BlockSpec((blk, 128), lambda i: (i, 0)),
        ],
        out_shape=[
            jax.ShapeDtypeStruct((B * N, H), jnp.float32),
            jax.ShapeDtypeStruct((B * N, H), jnp.float32),
            jax.ShapeDtypeStruct((B * N, 128), jnp.int32),
        ],
    )(hVf, W1a, W1c, E2)


_NW = 32
_NPW = N // _NW
_NBUF = 4


def _make_gather(b):
    mesh = plsc.VectorSubcoreMesh(core_axis_name="c", subcore_axis_name="s")

    @functools.partial(
        pl.kernel,
        out_type=jax.ShapeDtypeStruct((N, K, H), jnp.float32),
        mesh=mesh,
        scratch_types=[
            pltpu.VMEM((_NPW, 128), jnp.int32),
            pltpu.VMEM((_NBUF, K, H), jnp.float32),
            pltpu.SemaphoreType.DMA,
            pltpu.SemaphoreType.DMA,
        ],
    )
    def gather_kernel(q_hbm, idx_hbm, out_hbm, idx_v, rows_v, gsem, wsem):
        wid = lax.axis_index("s") * 2 + lax.axis_index("c")
        n0 = wid * _NPW
        pltpu.sync_copy(idx_hbm.at[pl.ds(b * N + n0, _NPW)], idx_v)

        gh = [None] * _NBUF
        wh = [None] * _NBUF
        la = _NBUF - 1
        for i in range(-la, _NPW):
            j = i + la
            if j < _NPW:
                bj = j % _NBUF
                if wh[bj] is not None:
                    wh[bj].wait()
                gh[bj] = pltpu.async_copy(q_hbm.at[idx_v.at[j, pl.ds(0, K)]],
                                          rows_v.at[bj], gsem)
            if i >= 0:
                bi = i % _NBUF
                gh[bi].wait()
                wh[bi] = pltpu.async_copy(rows_v.at[bi],
                                          out_hbm.at[n0 + i], wsem)
        for i in range(_NPW - _NBUF, _NPW):
            wh[i % _NBUF].wait()

    return gather_kernel


def _dot3(x, w):
    return lax.dot_general(x, w, (((x.ndim - 1,), (0,)), ((), ())),
                           preferred_element_type=jnp.float32)


def _main_body(prev_ref, hE_ref, g_ref, p_ref, hv_ref, ma_ref,
               w1b_ref, b1_ref, w2_ref, b2_ref, w3_ref, b3_ref,
               g1_ref, be1_ref, out_ref):
    del prev_ref
    x = hE_ref[...]
    e1 = _dot3(x, w1b_ref[...])
    pb = p_ref[...][:, None, :]
    t1 = _gelu(e1 + pb + g_ref[...] + b1_ref[...][None])
    t2 = _gelu(_dot3(t1, w2_ref[...]) + b2_ref[...][None])
    ma = ma_ref[...]
    t2m = t2 * ma[:, :, None]
    s = jnp.sum(t2m, axis=1)
    m = jnp.sum(ma, axis=1, keepdims=True)
    dh = (jnp.dot(s, w3_ref[...], preferred_element_type=jnp.float32)
          + m * b3_ref[...]) * _INV_K
    r = hv_ref[...] + dh
    mu = jnp.mean(r, axis=-1, keepdims=True)
    var = jnp.mean((r - mu) ** 2, axis=-1, keepdims=True)
    out_ref[...] = (r - mu) * lax.rsqrt(var + 1e-5) * g1_ref[...] + be1_ref[...]


def _main(b, prev, hE3, G_b, P, hVf, ma2, W1b, b1r, W2, b2r, W3, b3r,
          g1r, be1r):
    nblk = N // BN
    wspec = pl.BlockSpec((H, H), lambda i: (0, 0))
    bspec = pl.BlockSpec((1, H), lambda i: (0, 0))
    return pl.pallas_call(
        _main_body,
        grid=(nblk,),
        in_specs=[
            pl.BlockSpec(memory_space=pl.ANY),
            pl.BlockSpec((BN, K, H), lambda i: (b * nblk + i, 0, 0)),
            pl.BlockSpec((BN, K, H), lambda i: (i, 0, 0)),
            pl.BlockSpec((BN, H), lambda i: (b * nblk + i, 0)),
            pl.BlockSpec((BN, H), lambda i: (b * nblk + i, 0)),
            pl.BlockSpec((BN, K), lambda i: (b * nblk + i, 0)),
            wspec, bspec, wspec, bspec, wspec, bspec,
            bspec, bspec,
        ],
        out_specs=pl.BlockSpec((BN, H), lambda i: (b * nblk + i, 0)),
        out_shape=jax.ShapeDtypeStruct((B * N, H), jnp.float32),
        input_output_aliases={0: 0},
    )(prev, hE3, G_b, P, hVf, ma2, W1b, b1r, W2, b2r, W3, b3r, g1r, be1r)


def _ffn_body(x_ref, win_ref, bi_ref, wout_ref, bo_ref, g2_ref, be2_ref,
              mv_ref, out_ref):
    x = x_ref[...]
    t = _gelu(jnp.dot(x, win_ref[...], preferred_element_type=jnp.float32)
              + bi_ref[...])
    f = jnp.dot(t, wout_ref[...], preferred_element_type=jnp.float32) + bo_ref[...]
    r = x + f
    mu = jnp.mean(r, axis=-1, keepdims=True)
    var = jnp.mean((r - mu) ** 2, axis=-1, keepdims=True)
    out_ref[...] = ((r - mu) * lax.rsqrt(var + 1e-5) * g2_ref[...]
                    + be2_ref[...]) * mv_ref[...]


def _ffn(hv1, Win, bir, Wout, bor, g2r, be2r, mVf):
    blk = 512
    bspec = pl.BlockSpec((1, 4 * H), lambda i: (0, 0))
    return pl.pallas_call(
        _ffn_body,
        grid=(B * N // blk,),
        in_specs=[
            pl.BlockSpec((blk, H), lambda i: (i, 0)),
            pl.BlockSpec((H, 4 * H), lambda i: (0, 0)),
            bspec,
            pl.BlockSpec((4 * H, H), lambda i: (0, 0)),
            pl.BlockSpec((1, H), lambda i: (0, 0)),
            pl.BlockSpec((1, H), lambda i: (0, 0)),
            pl.BlockSpec((1, H), lambda i: (0, 0)),
            pl.BlockSpec((blk, 1), lambda i: (i, 0)),
        ],
        out_specs=pl.BlockSpec((blk, H), lambda i: (i, 0)),
        out_shape=jax.ShapeDtypeStruct((B * N, H), jnp.float32),
    )(hv1, Win, bir, Wout, bor, g2r, be2r, mVf)


def kernel(h_V, h_E, E_idx, mask_V, mask_attend, W1, b1, W2, b2, W3, b3,
           Win, bi, Wout, bo, g1, be1, g2, be2):
    hVf = h_V.reshape(B * N, H)
    W1a, W1b, W1c = W1[:H], W1[H:2 * H], W1[2 * H:]
    b1r, b2r, b3r = b1.reshape(1, H), b2.reshape(1, H), b3.reshape(1, H)
    g1r, be1r = g1.reshape(1, H), be1.reshape(1, H)
    g2r, be2r = g2.reshape(1, H), be2.reshape(1, H)
    bir, bor = bi.reshape(1, 4 * H), bo.reshape(1, H)

    P, Q, idxG = _project(hVf, W1a, W1c, E_idx.reshape(B * N, K))
    hE3 = h_E.reshape(B * N, K, H)
    ma2 = mask_attend.reshape(B * N, K)

    hv1 = jnp.zeros((B * N, H), jnp.float32)
    for b in range(B):
        G_b = _make_gather(b)(Q, idxG)
        hv1 = _main(b, hv1, hE3, G_b, P, hVf, ma2,
                    W1b, b1r, W2, b2r, W3, b3r, g1r, be1r)

    out = _ffn(hv1, Win, bir, Wout, bor, g2r, be2r, mask_V.reshape(B * N, 1))
    return out.reshape(B, N, H)


# speedup vs baseline: 1.0788x; 1.0011x over previous
import functools

import jax
import jax.numpy as jnp
from jax import lax
from jax.experimental import pallas as pl
from jax.experimental.pallas import tpu as pltpu
from jax.experimental.pallas import tpu_sc as plsc

B, N, K, H = 4, 1024, 36, 128
NB = N * K
BN = 32
EB = BN * K

_INV_K = 1.0 / 36.0
_SQRT_HALF = 0.7071067811865476


def _gelu(x):
    return x * 0.5 * (1.0 + lax.erf(x * _SQRT_HALF))


def _proj_body(hv_ref, w1a_ref, w1c_ref, e_ref, p_ref, q_ref, idx_ref):
    x = hv_ref[...]
    p_ref[...] = jnp.dot(x, w1a_ref[...], preferred_element_type=jnp.float32)
    q_ref[...] = jnp.dot(x, w1c_ref[...], preferred_element_type=jnp.float32)
    off = pl.program_id(0) * N
    ev = e_ref[...] + off
    idx_ref[...] = jnp.concatenate(
        [ev, jnp.zeros((ev.shape[0], 128 - K), jnp.int32)], axis=1)


def _project(hVf, W1a, W1c, E2):
    blk = 1024
    return pl.pallas_call(
        _proj_body,
        grid=(B * N // blk,),
        in_specs=[
            pl.BlockSpec((blk, H), lambda i: (i, 0)),
            pl.BlockSpec((H, H), lambda i: (0, 0)),
            pl.BlockSpec((H, H), lambda i: (0, 0)),
            pl.BlockSpec((blk, K), lambda i: (i, 0)),
        ],
        out_specs=[
            pl.BlockSpec((blk, H), lambda i: (i, 0)),
            pl.BlockSpec((blk, H), lambda i: (i, 0)),
            pl.BlockSpec((blk, 128), lambda i: (i, 0)),
        ],
        out_shape=[
            jax.ShapeDtypeStruct((B * N, H), jnp.float32),
            jax.ShapeDtypeStruct((B * N, H), jnp.float32),
            jax.ShapeDtypeStruct((B * N, 128), jnp.int32),
        ],
    )(hVf, W1a, W1c, E2)


_NW = 32
_NPW = N // _NW
_NBUF = 4


def _make_gather(b):
    mesh = plsc.VectorSubcoreMesh(core_axis_name="c", subcore_axis_name="s")

    @functools.partial(
        pl.kernel,
        out_type=jax.ShapeDtypeStruct((N, K, H), jnp.float32),
        mesh=mesh,
        scratch_types=[
            pltpu.VMEM((_NPW, 128), jnp.int32),
            pltpu.VMEM((_NBUF, K, H), jnp.float32),
            pltpu.SemaphoreType.DMA,
            pltpu.SemaphoreType.DMA,
        ],
        compiler_params=pltpu.CompilerParams(use_tc_tiling_on_sc=True),
    )
    def gather_kernel(q_hbm, idx_hbm, out_hbm, idx_v, rows_v, gsem, wsem):
        wid = lax.axis_index("s") * 2 + lax.axis_index("c")
        n0 = wid * _NPW
        pltpu.sync_copy(idx_hbm.at[pl.ds(b * N + n0, _NPW)], idx_v)

        gh = [None] * _NBUF
        wh = [None] * _NBUF
        la = _NBUF - 1
        for i in range(-la, _NPW):
            j = i + la
            if j < _NPW:
                bj = j % _NBUF
                if wh[bj] is not None:
                    wh[bj].wait()
                gh[bj] = pltpu.async_copy(q_hbm.at[idx_v.at[j, pl.ds(0, K)]],
                                          rows_v.at[bj], gsem)
            if i >= 0:
                bi = i % _NBUF
                gh[bi].wait()
                wh[bi] = pltpu.async_copy(rows_v.at[bi],
                                          out_hbm.at[n0 + i], wsem)
        for i in range(_NPW - _NBUF, _NPW):
            wh[i % _NBUF].wait()

    return gather_kernel


def _dot3(x, w):
    return lax.dot_general(x, w, (((x.ndim - 1,), (0,)), ((), ())),
                           preferred_element_type=jnp.float32)


def _main_body(prev_ref, hE_ref, g_ref, p_ref, hv_ref, ma_ref,
               w1b_ref, b1_ref, w2_ref, b2_ref, w3_ref, b3_ref,
               g1_ref, be1_ref, out_ref):
    del prev_ref
    x = hE_ref[...]
    e1 = _dot3(x, w1b_ref[...])
    pb = p_ref[...][:, None, :]
    t1 = _gelu(e1 + pb + g_ref[...] + b1_ref[...][None])
    t2 = _gelu(_dot3(t1, w2_ref[...]) + b2_ref[...][None])
    ma = ma_ref[...]
    t2m = t2 * ma[:, :, None]
    s = jnp.sum(t2m, axis=1)
    m = jnp.sum(ma, axis=1, keepdims=True)
    dh = (jnp.dot(s, w3_ref[...], preferred_element_type=jnp.float32)
          + m * b3_ref[...]) * _INV_K
    r = hv_ref[...] + dh
    mu = jnp.mean(r, axis=-1, keepdims=True)
    var = jnp.mean((r - mu) ** 2, axis=-1, keepdims=True)
    out_ref[...] = (r - mu) * lax.rsqrt(var + 1e-5) * g1_ref[...] + be1_ref[...]


def _main(b, prev, hE3, G_b, P, hVf, ma2, W1b, b1r, W2, b2r, W3, b3r,
          g1r, be1r):
    nblk = N // BN
    wspec = pl.BlockSpec((H, H), lambda i: (0, 0))
    bspec = pl.BlockSpec((1, H), lambda i: (0, 0))
    return pl.pallas_call(
        _main_body,
        grid=(nblk,),
        in_specs=[
            pl.BlockSpec(memory_space=pl.ANY),
            pl.BlockSpec((BN, K, H), lambda i: (b * nblk + i, 0, 0)),
            pl.BlockSpec((BN, K, H), lambda i: (i, 0, 0)),
            pl.BlockSpec((BN, H), lambda i: (b * nblk + i, 0)),
            pl.BlockSpec((BN, H), lambda i: (b * nblk + i, 0)),
            pl.BlockSpec((BN, K), lambda i: (b * nblk + i, 0)),
            wspec, bspec, wspec, bspec, wspec, bspec,
            bspec, bspec,
        ],
        out_specs=pl.BlockSpec((BN, H), lambda i: (b * nblk + i, 0)),
        out_shape=jax.ShapeDtypeStruct((B * N, H), jnp.float32),
        input_output_aliases={0: 0},
    )(prev, hE3, G_b, P, hVf, ma2, W1b, b1r, W2, b2r, W3, b3r, g1r, be1r)


def _ffn_body(x_ref, win_ref, bi_ref, wout_ref, bo_ref, g2_ref, be2_ref,
              mv_ref, out_ref):
    x = x_ref[...]
    t = _gelu(jnp.dot(x, win_ref[...], preferred_element_type=jnp.float32)
              + bi_ref[...])
    f = jnp.dot(t, wout_ref[...], preferred_element_type=jnp.float32) + bo_ref[...]
    r = x + f
    mu = jnp.mean(r, axis=-1, keepdims=True)
    var = jnp.mean((r - mu) ** 2, axis=-1, keepdims=True)
    out_ref[...] = ((r - mu) * lax.rsqrt(var + 1e-5) * g2_ref[...]
                    + be2_ref[...]) * mv_ref[...]


def _ffn(hv1, Win, bir, Wout, bor, g2r, be2r, mVf):
    blk = 512
    bspec = pl.BlockSpec((1, 4 * H), lambda i: (0, 0))
    return pl.pallas_call(
        _ffn_body,
        grid=(B * N // blk,),
        in_specs=[
            pl.BlockSpec((blk, H), lambda i: (i, 0)),
            pl.BlockSpec((H, 4 * H), lambda i: (0, 0)),
            bspec,
            pl.BlockSpec((4 * H, H), lambda i: (0, 0)),
            pl.BlockSpec((1, H), lambda i: (0, 0)),
            pl.BlockSpec((1, H), lambda i: (0, 0)),
            pl.BlockSpec((1, H), lambda i: (0, 0)),
            pl.BlockSpec((blk, 1), lambda i: (i, 0)),
        ],
        out_specs=pl.BlockSpec((blk, H), lambda i: (i, 0)),
        out_shape=jax.ShapeDtypeStruct((B * N, H), jnp.float32),
    )(hv1, Win, bir, Wout, bor, g2r, be2r, mVf)


def kernel(h_V, h_E, E_idx, mask_V, mask_attend, W1, b1, W2, b2, W3, b3,
           Win, bi, Wout, bo, g1, be1, g2, be2):
    hVf = h_V.reshape(B * N, H)
    W1a, W1b, W1c = W1[:H], W1[H:2 * H], W1[2 * H:]
    b1r, b2r, b3r = b1.reshape(1, H), b2.reshape(1, H), b3.reshape(1, H)
    g1r, be1r = g1.reshape(1, H), be1.reshape(1, H)
    g2r, be2r = g2.reshape(1, H), be2.reshape(1, H)
    bir, bor = bi.reshape(1, 4 * H), bo.reshape(1, H)

    P, Q, idxG = _project(hVf, W1a, W1c, E_idx.reshape(B * N, K))
    hE3 = h_E.reshape(B * N, K, H)
    ma2 = mask_attend.reshape(B * N, K)

    hv1 = jnp.zeros((B * N, H), jnp.float32)
    for b in range(B):
        G_b = _make_gather(b)(Q, idxG)
        hv1 = _main(b, hv1, hE3, G_b, P, hVf, ma2,
                    W1b, b1r, W2, b2r, W3, b3r, g1r, be1r)

    out = _ffn(hv1, Win, bir, Wout, bor, g2r, be2r, mask_V.reshape(B * N, 1))
    return out.reshape(B, N, H)
